# Initial kernel scaffold; baseline (speedup 1.0000x reference)
import jax
import jax.numpy as jnp
from jax.experimental import pallas as pl


def kernel(query, k, keys, vals):
    raise NotImplementedError("write your pallas kernel here")


# speedup vs baseline: 2.9952x; 2.9952x over previous
import functools

import jax
import jax.numpy as jnp
from jax.experimental import pallas as pl

K_TOP = 16
BB = 256
NB = 2048
S = 16


def _norm_kernel(x_ref, o_ref):
    x = x_ref[...]
    n = jnp.sqrt(jnp.sum(x * x, axis=-1, keepdims=True))
    o_ref[...] = x / jnp.maximum(n, 1e-12)


def _l2norm_rows(x, rows_per_block):
    r, d = x.shape
    return pl.pallas_call(
        _norm_kernel,
        grid=(r // rows_per_block,),
        in_specs=[pl.BlockSpec((rows_per_block, d), lambda i: (i, 0))],
        out_specs=pl.BlockSpec((rows_per_block, d), lambda i: (i, 0)),
        out_shape=jax.ShapeDtypeStruct((r, d), jnp.float32),
    )(x)


def _sim_kernel(q_ref, k_ref, sim_ref, bm_ref, *, nvalid):
    t = pl.program_id(1)
    q = q_ref[...]
    kt = k_ref[...]
    sim = jax.lax.dot_general(
        q, kt, (((1,), (1,)), ((), ())), preferred_element_type=jnp.float32)
    limit = nvalid - t * NB
    col = jax.lax.broadcasted_iota(jnp.int32, (BB, NB), 1)
    sim = jnp.where(col < limit, sim, -1e30)
    sim_ref[...] = sim
    bm_ref[...] = jnp.max(sim.reshape(BB, S, NB // S), axis=1)


def kernel(query, k, keys, vals):
    B, D = query.shape
    n = keys.shape[0]
    npad = ((n + NB - 1) // NB) * NB
    nblocks = npad // S

    keys_p = jnp.pad(keys, ((0, npad - n), (0, 0)))
    qn = _l2norm_rows(query, 512)
    kn = _l2norm_rows(keys_p, NB)

    sim, bmax = pl.pallas_call(
        functools.partial(_sim_kernel, nvalid=n),
        grid=(B // BB, npad // NB),
        in_specs=[
            pl.BlockSpec((BB, D), lambda b, t: (b, 0)),
            pl.BlockSpec((NB, D), lambda b, t: (t, 0)),
        ],
        out_specs=[
            pl.BlockSpec((BB, NB), lambda b, t: (b, t)),
            pl.BlockSpec((BB, NB // S), lambda b, t: (b, t)),
        ],
        out_shape=[
            jax.ShapeDtypeStruct((B, npad), jnp.float32),
            jax.ShapeDtypeStruct((B, nblocks), jnp.float32),
        ],
    )(qn, kn)

    _, bids = jax.lax.top_k(bmax, K_TOP)
    tchunk = bids // (NB // S)
    boff = bids % (NB // S)
    s = jnp.arange(S, dtype=jnp.int32)
    cand_cols = (tchunk * NB + boff)[:, :, None] + (NB // S) * s[None, None, :]
    cand_cols = cand_cols.reshape(B, K_TOP * S)
    cand_vals = jnp.take_along_axis(sim, cand_cols, axis=1)
    _, cpos = jax.lax.top_k(cand_vals, K_TOP)
    idx = jnp.take_along_axis(cand_cols, cpos, axis=1).astype(jnp.int32)

    out_keys = jnp.take(keys, idx, axis=0)
    out_vals = jnp.take(vals, idx, axis=0)
    scores = jnp.zeros((B, K_TOP), dtype=jnp.float32)
    return (out_keys, out_vals, scores, idx)


# speedup vs baseline: 4.4303x; 1.4791x over previous
import functools

import jax
import jax.numpy as jnp
from jax import lax
from jax.experimental import pallas as pl
from jax.experimental.pallas import tpu as pltpu
from jax.experimental.pallas import tpu_sc as plsc

K_TOP = 16
BB = 256
NB = 2048
S = 16
L = 16
NEG = -1e30


def _norm_kernel(x_ref, o_ref):
    x = x_ref[...]
    n = jnp.sqrt(jnp.sum(x * x, axis=-1, keepdims=True))
    o_ref[...] = x / jnp.maximum(n, 1e-12)


def _l2norm_rows(x, rows_per_block):
    r, d = x.shape
    return pl.pallas_call(
        _norm_kernel,
        grid=(r // rows_per_block,),
        in_specs=[pl.BlockSpec((rows_per_block, d), lambda i: (i, 0))],
        out_specs=pl.BlockSpec((rows_per_block, d), lambda i: (i, 0)),
        out_shape=jax.ShapeDtypeStruct((r, d), jnp.float32),
    )(x)


def _sim_kernel(q_ref, k_ref, sim_ref, bm_ref, *, nvalid):
    t = pl.program_id(1)
    q = q_ref[...]
    kt = k_ref[...]
    sim = jax.lax.dot_general(
        q, kt, (((1,), (1,)), ((), ())), preferred_element_type=jnp.float32)
    limit = nvalid - t * NB
    col = jax.lax.broadcasted_iota(jnp.int32, (BB, NB), 1)
    sim = jnp.where(col < limit, sim, NEG)
    sim_ref[...] = sim
    bm_ref[...] = jnp.max(sim.reshape(BB, S, NB // S), axis=1)


def _merge16(bv, bi, v, ids):
    vd, idd = plsc.sort_key_val(v, ids, descending=True)
    take = vd > bv
    mv = jnp.where(take, vd, bv)
    mi = jnp.where(take, idd, bi)
    return tuple(plsc.sort_key_val(mv, mi))


def _make_sc_topk(B, npad, n, rows_per_worker, num_cores, num_subcores):
    nblocks = npad // S
    nvec = nblocks // L
    ngrp = nvec // 4
    spc = NB // S
    d = 128
    mesh = plsc.VectorSubcoreMesh(
        core_axis_name="c", subcore_axis_name="s")

    @functools.partial(
        pl.kernel,
        out_type=[
            jax.ShapeDtypeStruct((B, K_TOP), jnp.int32),
            jax.ShapeDtypeStruct((B, K_TOP, d), jnp.float32),
            jax.ShapeDtypeStruct((B, K_TOP, d), jnp.float32),
        ],
        mesh=mesh,
        scratch_types=[
            pltpu.VMEM((2 * nblocks,), jnp.float32),
            pltpu.VMEM((nblocks + L,), jnp.float32),
            pltpu.VMEM((nblocks + L,), jnp.int32),
            pltpu.VMEM((128,), jnp.int32),
            pltpu.VMEM((128,), jnp.int32),
            pltpu.VMEM((128, L), jnp.float32),
            pltpu.VMEM((128, L), jnp.float32),
            pltpu.VMEM((K_TOP,), jnp.int32),
            pltpu.VMEM((K_TOP, d), jnp.float32),
            pltpu.VMEM((K_TOP, d), jnp.float32),
            pltpu.SMEM((1,), jnp.int32),
            pltpu.SemaphoreType.DMA,
            pltpu.SemaphoreType.DMA,
            pltpu.SemaphoreType.DMA,
            pltpu.SemaphoreType.DMA,
            pltpu.SemaphoreType.DMA,
        ],
        compiler_params=pltpu.CompilerParams(
            needs_layout_passes=False, use_tc_tiling_on_sc=False),
    )
    def sc_topk(bm_hbm, sim2_hbm, keys_hbm, vals_hbm,
                idx_hbm, outk_hbm, outv_hbm,
                bm_v, cv_v, ci_v, ga_v, gb_v, cra_v, crb_v,
                ixs_v, kr_v, vr_v, cnt_s, sem, sem2, sem3, sem4, sem5):
        wid = lax.axis_index("s") * num_cores + lax.axis_index("c")
        row0 = wid * rows_per_worker
        iota = lax.iota(jnp.int32, L)

        pltpu.async_copy(bm_hbm.at[row0], bm_v.at[pl.ds(0, nblocks)], sem).wait()

        def row_body(rl, _):
            r = row0 + rl
            off = (rl % 2) * nblocks
            noff = ((rl + 1) % 2) * nblocks
            @pl.when(rl + 1 < rows_per_worker)
            def _():
                pltpu.async_copy(bm_hbm.at[r + 1],
                                 bm_v.at[pl.ds(noff, nblocks)], sem2)

            def amax_body(g, m):
                for t in range(4):
                    m = jnp.maximum(m, bm_v[pl.ds(off + g * 64 + t * L, L)])
                return m
            m = lax.fori_loop(0, ngrp, amax_body, jnp.full((L,), NEG))
            t0 = jnp.min(m)

            cnt_s[0] = 0

            def collect_body(g, _):
                mx = bm_v[pl.ds(off + g * 64, L)]
                for t in range(1, 4):
                    mx = jnp.maximum(mx, bm_v[pl.ds(off + g * 64 + t * L, L)])

                @pl.when(jnp.max(mx) >= t0)
                def _():
                    for t in range(4):
                        v = bm_v[pl.ds(off + g * 64 + t * L, L)]
                        msk = v >= t0
                        mi = msk.astype(jnp.int32)
                        incl = plsc.cumsum(mi)
                        pos = cnt_s[0] + incl - mi
                        ids = (g * 4 + t) * L + iota
                        plsc.store_scatter(cv_v, [pos], v, mask=msk)
                        plsc.store_scatter(ci_v, [pos], ids, mask=msk)
                        cnt_s[0] = cnt_s[0] + jnp.max(incl)
                return 0

            lax.fori_loop(0, ngrp, collect_body, 0)
            cnt = cnt_s[0]
            plsc.store_scatter(cv_v, [cnt + iota], jnp.full((L,), NEG))
            plsc.store_scatter(ci_v, [cnt + iota], iota)

            bv, bi = plsc.sort_key_val(cv_v[pl.ds(0, L)], ci_v[pl.ds(0, L)])
            nchunks = (cnt + L - 1) // L

            def bmerge_body(c, carry):
                bv, bi = carry
                return _merge16(bv, bi, cv_v[pl.ds(c * L, L)],
                                ci_v[pl.ds(c * L, L)])
            bv, bi = lax.fori_loop(1, nchunks, bmerge_body, (bv, bi))

            tchunk = bi // spc
            boff = bi % spc
            base_g = r * (npad // L) + tchunk * spc + boff // L
            lane = boff % L
            colbase = tchunk * NB + boff
            for s in range(8):
                ga_v[pl.ds(s * L, L)] = base_g + 8 * s
                gb_v[pl.ds(s * L, L)] = base_g + 8 * (s + 8)
            cpa = pltpu.async_copy(sim2_hbm.at[ga_v], cra_v, sem)
            cpb = pltpu.async_copy(sim2_hbm.at[gb_v], crb_v, sem3)
            cpa.wait()
            cpb.wait()

            ev = plsc.load_gather(cra_v, [iota, lane])
            ei = colbase
            ev, ei = plsc.sort_key_val(ev, ei)
            for s in range(1, 16):
                src = cra_v if s < 8 else crb_v
                rowv = (s % 8) * L + iota
                v = plsc.load_gather(src, [rowv, lane])
                ev, ei = _merge16(ev, ei, v, colbase + 128 * s)

            ei_d = lax.rev(ei, (0,))
            ixs_v[...] = ei_d
            cpk = pltpu.async_copy(keys_hbm.at[ei_d], kr_v, sem4)
            cpv = pltpu.async_copy(vals_hbm.at[ei_d], vr_v, sem5)
            pltpu.sync_copy(ixs_v, idx_hbm.at[r])
            cpk.wait()
            cpv.wait()
            pltpu.sync_copy(kr_v, outk_hbm.at[r])
            pltpu.sync_copy(vr_v, outv_hbm.at[r])

            @pl.when(rl + 1 < rows_per_worker)
            def _():
                pltpu.make_async_copy(
                    bm_hbm.at[r + 1], bm_v.at[pl.ds(noff, nblocks)],
                    sem2).wait()
            return 0

        lax.fori_loop(0, rows_per_worker, row_body, 0)

    return sc_topk


def kernel(query, k, keys, vals):
    B, D = query.shape
    n = keys.shape[0]
    npad = ((n + NB - 1) // NB) * NB
    nblocks = npad // S

    keys_p = jnp.pad(keys, ((0, npad - n), (0, 0)))
    qn = _l2norm_rows(query, 512)
    kn = _l2norm_rows(keys_p, NB)

    sim, bmax = pl.pallas_call(
        functools.partial(_sim_kernel, nvalid=n),
        grid=(B // BB, npad // NB),
        in_specs=[
            pl.BlockSpec((BB, D), lambda b, t: (b, 0)),
            pl.BlockSpec((NB, D), lambda b, t: (t, 0)),
        ],
        out_specs=[
            pl.BlockSpec((BB, NB), lambda b, t: (b, t)),
            pl.BlockSpec((BB, NB // S), lambda b, t: (b, t)),
        ],
        out_shape=[
            jax.ShapeDtypeStruct((B, npad), jnp.float32),
            jax.ShapeDtypeStruct((B, nblocks), jnp.float32),
        ],
    )(qn, kn)

    num_cores, num_subcores = 2, 16
    nw = num_cores * num_subcores
    sc = _make_sc_topk(B, npad, n, B // nw, num_cores, num_subcores)
    sim2 = sim.reshape(B * npad // L, L)
    idx, out_keys, out_vals = sc(bmax, sim2, keys, vals)
    scores = jnp.zeros((B, K_TOP), dtype=jnp.float32)
    return (out_keys, out_vals, scores, idx)


# speedup vs baseline: 4.6321x; 1.0456x over previous
import functools

import jax
import jax.numpy as jnp
from jax import lax
from jax.experimental import pallas as pl
from jax.experimental.pallas import tpu as pltpu
from jax.experimental.pallas import tpu_sc as plsc

K_TOP = 16
BB = 1024
CH = 128
SG = 16
L = 16
NEG = -1e30


def _norm_kernel(x_ref, o_ref):
    x = x_ref[...]
    n = jnp.sqrt(jnp.sum(x * x, axis=-1, keepdims=True))
    o_ref[...] = x / jnp.maximum(n, 1e-12)


def _l2norm_rows(x, rows_per_block):
    r, d = x.shape
    return pl.pallas_call(
        _norm_kernel,
        grid=(r // rows_per_block,),
        in_specs=[pl.BlockSpec((rows_per_block, d), lambda i: (i, 0))],
        out_specs=pl.BlockSpec((rows_per_block, d), lambda i: (i, 0)),
        out_shape=jax.ShapeDtypeStruct((r, d), jnp.float32),
    )(x)


def _sim_kernel(q_ref, k_ref, sim_ref, bm_ref, *, nvalid):
    v = pl.program_id(1)
    q = q_ref[...]
    kt = k_ref[...]
    sim = jax.lax.dot_general(
        q, kt, (((1,), (1,)), ((), ())), preferred_element_type=jnp.float32)
    limit = nvalid - v * CH
    col = jax.lax.broadcasted_iota(jnp.int32, (BB, CH), 1)
    sim = jnp.where(col < limit, sim, NEG)
    sim_ref[...] = sim.reshape(1, BB, CH)
    bm4 = sim.reshape(BB // 8, 1, 8, CH)

    @pl.when(v % SG == 0)
    def _():
        bm_ref[...] = bm4

    @pl.when(v % SG != 0)
    def _():
        bm_ref[...] = jnp.maximum(bm_ref[...], bm4)


def _merge16(bv, bi, v, ids):
    vd, idd = plsc.sort_key_val(v, ids, descending=True)
    take = vd > bv
    mv = jnp.where(take, vd, bv)
    mi = jnp.where(take, idd, bi)
    return tuple(plsc.sort_key_val(mv, mi))


def _make_sc_topk(B, npad, n, rows_per_worker, num_cores, num_subcores):
    nt = npad // (CH * SG)
    nblocks = nt * CH
    d = 128
    mesh = plsc.VectorSubcoreMesh(
        core_axis_name="c", subcore_axis_name="s")

    @functools.partial(
        pl.kernel,
        out_type=[
            jax.ShapeDtypeStruct((B, K_TOP), jnp.int32),
            jax.ShapeDtypeStruct((B, K_TOP, d), jnp.float32),
            jax.ShapeDtypeStruct((B, K_TOP, d), jnp.float32),
        ],
        mesh=mesh,
        scratch_types=[
            pltpu.VMEM((2 * nt, CH), jnp.float32),
            pltpu.VMEM((nblocks + L,), jnp.float32),
            pltpu.VMEM((nblocks + L,), jnp.int32),
            pltpu.VMEM((128,), jnp.int32),
            pltpu.VMEM((128,), jnp.int32),
            pltpu.VMEM((128, L), jnp.float32),
            pltpu.VMEM((128, L), jnp.float32),
            pltpu.VMEM((K_TOP,), jnp.int32),
            pltpu.VMEM((K_TOP, d), jnp.float32),
            pltpu.VMEM((K_TOP, d), jnp.float32),
            pltpu.SMEM((1,), jnp.int32),
            pltpu.SemaphoreType.DMA,
            pltpu.SemaphoreType.DMA,
            pltpu.SemaphoreType.DMA,
            pltpu.SemaphoreType.DMA,
            pltpu.SemaphoreType.DMA,
        ],
        compiler_params=pltpu.CompilerParams(
            needs_layout_passes=False, use_tc_tiling_on_sc=False),
    )
    def sc_topk(bm_hbm, sim2_hbm, keys_hbm, vals_hbm,
                idx_hbm, outk_hbm, outv_hbm,
                bm_v, cv_v, ci_v, ga_v, gb_v, cra_v, crb_v,
                ixs_v, kr_v, vr_v, cnt_s, sem, sem2, sem3, sem4, sem5):
        wid = lax.axis_index("s") * num_cores + lax.axis_index("c")
        row0 = wid * rows_per_worker
        iota = lax.iota(jnp.int32, L)

        pltpu.async_copy(bm_hbm.at[row0 // 8, :, row0 % 8, :],
                         bm_v.at[pl.ds(0, nt), :], sem).wait()

        def row_body(rl, _):
            r = row0 + rl
            toff = (rl % 2) * nt
            ntoff = ((rl + 1) % 2) * nt
            @pl.when(rl + 1 < rows_per_worker)
            def _():
                pltpu.async_copy(bm_hbm.at[(r + 1) // 8, :, (r + 1) % 8, :],
                                 bm_v.at[pl.ds(ntoff, nt), :], sem2)

            def amax_body(t, m):
                for i in range(8):
                    m = jnp.maximum(m, bm_v[toff + t, pl.ds(i * L, L)])
                return m
            m = lax.fori_loop(0, nt, amax_body, jnp.full((L,), NEG))
            t0 = jnp.min(m)

            cnt_s[0] = 0

            def collect_body(t, _):
                mx = bm_v[toff + t, pl.ds(0, L)]
                for i in range(1, 8):
                    mx = jnp.maximum(mx, bm_v[toff + t, pl.ds(i * L, L)])

                @pl.when(jnp.max(mx) >= t0)
                def _():
                    for i in range(8):
                        v = bm_v[toff + t, pl.ds(i * L, L)]
                        msk = v >= t0
                        mi = msk.astype(jnp.int32)
                        incl = plsc.cumsum(mi)
                        pos = cnt_s[0] + incl - mi
                        ids = t * CH + i * L + iota
                        plsc.store_scatter(cv_v, [pos], v, mask=msk)
                        plsc.store_scatter(ci_v, [pos], ids, mask=msk)
                        cnt_s[0] = cnt_s[0] + jnp.max(incl)
                return 0

            lax.fori_loop(0, nt, collect_body, 0)
            cnt = cnt_s[0]
            plsc.store_scatter(cv_v, [cnt + iota], jnp.full((L,), NEG))
            plsc.store_scatter(ci_v, [cnt + iota], iota)

            bv, bi = plsc.sort_key_val(cv_v[pl.ds(0, L)], ci_v[pl.ds(0, L)])
            nchunks = (cnt + L - 1) // L

            def bmerge_body(c, carry):
                bv, bi = carry
                return _merge16(bv, bi, cv_v[pl.ds(c * L, L)],
                                ci_v[pl.ds(c * L, L)])
            bv, bi = lax.fori_loop(1, nchunks, bmerge_body, (bv, bi))

            tchunk = bi // CH
            boff = bi % CH
            sstride = B * 8
            base_g = (tchunk * SG) * sstride + r * 8 + boff // L
            lane = boff % L
            colbase = tchunk * (CH * SG) + boff
            for s in range(8):
                ga_v[pl.ds(s * L, L)] = base_g + sstride * s
                gb_v[pl.ds(s * L, L)] = base_g + sstride * (s + 8)
            cpa = pltpu.async_copy(sim2_hbm.at[ga_v], cra_v, sem)
            cpb = pltpu.async_copy(sim2_hbm.at[gb_v], crb_v, sem3)
            cpa.wait()
            cpb.wait()

            ev = plsc.load_gather(cra_v, [iota, lane])
            ei = colbase
            ev, ei = plsc.sort_key_val(ev, ei)
            for s in range(1, 16):
                src = cra_v if s < 8 else crb_v
                rowv = (s % 8) * L + iota
                v = plsc.load_gather(src, [rowv, lane])
                ev, ei = _merge16(ev, ei, v, colbase + CH * s)

            ei_d = lax.rev(ei, (0,))
            ixs_v[...] = ei_d
            cpk = pltpu.async_copy(keys_hbm.at[ei_d], kr_v, sem4)
            cpv = pltpu.async_copy(vals_hbm.at[ei_d], vr_v, sem5)
            pltpu.sync_copy(ixs_v, idx_hbm.at[r])
            cpk.wait()
            cpv.wait()
            pltpu.sync_copy(kr_v, outk_hbm.at[r])
            pltpu.sync_copy(vr_v, outv_hbm.at[r])

            @pl.when(rl + 1 < rows_per_worker)
            def _():
                pltpu.make_async_copy(
                    bm_hbm.at[(r + 1) // 8, :, (r + 1) % 8, :],
                    bm_v.at[pl.ds(ntoff, nt), :], sem2).wait()
            return 0

        lax.fori_loop(0, rows_per_worker, row_body, 0)

    return sc_topk


def kernel(query, k, keys, vals):
    B, D = query.shape
    n = keys.shape[0]
    npad = ((n + CH * SG - 1) // (CH * SG)) * (CH * SG)
    nv = npad // CH

    keys_p = jnp.pad(keys, ((0, npad - n), (0, 0)))
    qn = _l2norm_rows(query, 512)
    kn = _l2norm_rows(keys_p, 2048)

    simv, bmax = pl.pallas_call(
        functools.partial(_sim_kernel, nvalid=n),
        grid=(B // BB, nv),
        in_specs=[
            pl.BlockSpec((BB, D), lambda b, v: (b, 0)),
            pl.BlockSpec((CH, D), lambda b, v: (v, 0)),
        ],
        out_specs=[
            pl.BlockSpec((1, BB, CH), lambda b, v: (v, b, 0)),
            pl.BlockSpec((BB // 8, 1, 8, CH), lambda b, v: (b, v // SG, 0, 0)),
        ],
        out_shape=[
            jax.ShapeDtypeStruct((nv, B, CH), jnp.float32),
            jax.ShapeDtypeStruct((B // 8, nv // SG, 8, CH), jnp.float32),
        ],
    )(qn, kn)

    num_cores, num_subcores = 2, 16
    nw = num_cores * num_subcores
    sc = _make_sc_topk(B, npad, n, B // nw, num_cores, num_subcores)
    sim2 = simv.reshape(B * npad // L, L)
    idx, out_keys, out_vals = sc(bmax, sim2, keys, vals)
    scores = jnp.zeros((B, K_TOP), dtype=jnp.float32)
    return (out_keys, out_vals, scores, idx)


# speedup vs baseline: 6.5617x; 1.4166x over previous
import functools

import jax
import jax.numpy as jnp
from jax import lax
from jax.experimental import pallas as pl
from jax.experimental.pallas import tpu as pltpu
from jax.experimental.pallas import tpu_sc as plsc

K_TOP = 16
BB = 256
CH = 128
SG = 16
L = 16
NEG = -1e30


def _norm_kernel(x_ref, o_ref):
    x = x_ref[...]
    n = jnp.sqrt(jnp.sum(x * x, axis=-1, keepdims=True))
    o_ref[...] = x / jnp.maximum(n, 1e-12)


def _l2norm_rows(x, rows_per_block):
    r, d = x.shape
    return pl.pallas_call(
        _norm_kernel,
        grid=(r // rows_per_block,),
        in_specs=[pl.BlockSpec((rows_per_block, d), lambda i: (i, 0))],
        out_specs=pl.BlockSpec((rows_per_block, d), lambda i: (i, 0)),
        out_shape=jax.ShapeDtypeStruct((r, d), jnp.float32),
    )(x)


def _sim_kernel(q_ref, k_ref, sim_ref, bm_ref, *, nvalid):
    t = pl.program_id(1)
    nb = CH * SG
    q = q_ref[...]
    kt = k_ref[...]
    sim = jax.lax.dot_general(
        q, kt, (((1,), (1,)), ((), ())), preferred_element_type=jnp.float32)
    limit = nvalid - t * nb
    col = jax.lax.broadcasted_iota(jnp.int32, (BB, nb), 1)
    sim = jnp.where(col < limit, sim, NEG)
    sim_ref[...] = sim.reshape(BB // 8, 8, SG, CH).swapaxes(1, 2)
    bm_ref[...] = jnp.max(sim.reshape(BB, SG, CH), axis=1).reshape(
        BB // 8, 1, 8, CH)


def _merge16(bv, bi, v, ids):
    vd, idd = plsc.sort_key_val(v, ids, descending=True)
    take = vd > bv
    mv = jnp.where(take, vd, bv)
    mi = jnp.where(take, idd, bi)
    return tuple(plsc.sort_key_val(mv, mi))


def _make_sc_topk(B, npad, n, rows_per_worker, num_cores, num_subcores):
    nt = npad // (CH * SG)
    nblocks = nt * CH
    d = 128
    mesh = plsc.VectorSubcoreMesh(
        core_axis_name="c", subcore_axis_name="s")

    @functools.partial(
        pl.kernel,
        out_type=[
            jax.ShapeDtypeStruct((B, K_TOP), jnp.int32),
            jax.ShapeDtypeStruct((B, K_TOP, d), jnp.float32),
            jax.ShapeDtypeStruct((B, K_TOP, d), jnp.float32),
        ],
        mesh=mesh,
        scratch_types=[
            pltpu.VMEM((2 * nt, CH), jnp.float32),
            pltpu.VMEM((nblocks + L,), jnp.float32),
            pltpu.VMEM((nblocks + L,), jnp.int32),
            pltpu.VMEM((128,), jnp.int32),
            pltpu.VMEM((128,), jnp.int32),
            pltpu.VMEM((128, L), jnp.float32),
            pltpu.VMEM((128, L), jnp.float32),
            pltpu.VMEM((K_TOP,), jnp.int32),
            pltpu.VMEM((K_TOP, d), jnp.float32),
            pltpu.VMEM((K_TOP, d), jnp.float32),
            pltpu.SMEM((1,), jnp.int32),
            pltpu.SemaphoreType.DMA,
            pltpu.SemaphoreType.DMA,
            pltpu.SemaphoreType.DMA,
            pltpu.SemaphoreType.DMA,
            pltpu.SemaphoreType.DMA,
        ],
        compiler_params=pltpu.CompilerParams(
            needs_layout_passes=False, use_tc_tiling_on_sc=False),
    )
    def sc_topk(bm_hbm, sim2_hbm, keys_hbm, vals_hbm,
                idx_hbm, outk_hbm, outv_hbm,
                bm_v, cv_v, ci_v, ga_v, gb_v, cra_v, crb_v,
                ixs_v, kr_v, vr_v, cnt_s, sem, sem2, sem3, sem4, sem5):
        wid = lax.axis_index("s") * num_cores + lax.axis_index("c")
        row0 = wid * rows_per_worker
        iota = lax.iota(jnp.int32, L)

        pltpu.async_copy(bm_hbm.at[row0 // 8, :, row0 % 8, :],
                         bm_v.at[pl.ds(0, nt), :], sem).wait()

        def row_body(rl, _):
            r = row0 + rl
            toff = (rl % 2) * nt
            ntoff = ((rl + 1) % 2) * nt
            @pl.when(rl + 1 < rows_per_worker)
            def _():
                pltpu.async_copy(bm_hbm.at[(r + 1) // 8, :, (r + 1) % 8, :],
                                 bm_v.at[pl.ds(ntoff, nt), :], sem2)

            def amax_body(t, m):
                for i in range(8):
                    m = jnp.maximum(m, bm_v[toff + t, pl.ds(i * L, L)])
                return m
            m = lax.fori_loop(0, nt, amax_body, jnp.full((L,), NEG))
            t0 = jnp.min(m)

            cnt_s[0] = 0

            def collect_body(t, _):
                mx = bm_v[toff + t, pl.ds(0, L)]
                for i in range(1, 8):
                    mx = jnp.maximum(mx, bm_v[toff + t, pl.ds(i * L, L)])

                @pl.when(jnp.max(mx) >= t0)
                def _():
                    for i in range(8):
                        v = bm_v[toff + t, pl.ds(i * L, L)]
                        msk = v >= t0
                        mi = msk.astype(jnp.int32)
                        incl = plsc.cumsum(mi)
                        pos = cnt_s[0] + incl - mi
                        ids = t * CH + i * L + iota
                        plsc.store_scatter(cv_v, [pos], v, mask=msk)
                        plsc.store_scatter(ci_v, [pos], ids, mask=msk)
                        cnt_s[0] = cnt_s[0] + jnp.max(incl)
                return 0

            lax.fori_loop(0, nt, collect_body, 0)
            cnt = cnt_s[0]
            plsc.store_scatter(cv_v, [cnt + iota], jnp.full((L,), NEG))
            plsc.store_scatter(ci_v, [cnt + iota], iota)

            bv, bi = plsc.sort_key_val(cv_v[pl.ds(0, L)], ci_v[pl.ds(0, L)])
            nchunks = (cnt + L - 1) // L

            def bmerge_body(c, carry):
                bv, bi = carry
                return _merge16(bv, bi, cv_v[pl.ds(c * L, L)],
                                ci_v[pl.ds(c * L, L)])
            bv, bi = lax.fori_loop(1, nchunks, bmerge_body, (bv, bi))

            tchunk = bi // CH
            boff = bi % CH
            base_g = ((r // 8) * (npad // L) + tchunk * (SG * 64)
                      + (r % 8) * 8 + boff // L)
            lane = boff % L
            colbase = tchunk * (CH * SG) + boff
            for s in range(8):
                ga_v[pl.ds(s * L, L)] = base_g + 64 * s
                gb_v[pl.ds(s * L, L)] = base_g + 64 * (s + 8)
            cpa = pltpu.async_copy(sim2_hbm.at[ga_v], cra_v, sem)
            cpb = pltpu.async_copy(sim2_hbm.at[gb_v], crb_v, sem3)
            cpa.wait()
            cpb.wait()

            ev = plsc.load_gather(cra_v, [iota, lane])
            ei = colbase
            ev, ei = plsc.sort_key_val(ev, ei)
            for s in range(1, 16):
                src = cra_v if s < 8 else crb_v
                rowv = (s % 8) * L + iota
                v = plsc.load_gather(src, [rowv, lane])
                ev, ei = _merge16(ev, ei, v, colbase + CH * s)

            ei_d = lax.rev(ei, (0,))
            ixs_v[...] = ei_d
            cpk = pltpu.async_copy(keys_hbm.at[ei_d], kr_v, sem4)
            cpv = pltpu.async_copy(vals_hbm.at[ei_d], vr_v, sem5)
            pltpu.sync_copy(ixs_v, idx_hbm.at[r])
            cpk.wait()
            cpv.wait()
            pltpu.sync_copy(kr_v, outk_hbm.at[r])
            pltpu.sync_copy(vr_v, outv_hbm.at[r])

            @pl.when(rl + 1 < rows_per_worker)
            def _():
                pltpu.make_async_copy(
                    bm_hbm.at[(r + 1) // 8, :, (r + 1) % 8, :],
                    bm_v.at[pl.ds(ntoff, nt), :], sem2).wait()
            return 0

        lax.fori_loop(0, rows_per_worker, row_body, 0)

    return sc_topk


def kernel(query, k, keys, vals):
    B, D = query.shape
    n = keys.shape[0]
    npad = ((n + CH * SG - 1) // (CH * SG)) * (CH * SG)
    nv = npad // CH

    keys_p = jnp.pad(keys, ((0, npad - n), (0, 0)))
    qn = _l2norm_rows(query, 512)
    kn = _l2norm_rows(keys_p, 2048)

    simv, bmax = pl.pallas_call(
        functools.partial(_sim_kernel, nvalid=n),
        grid=(B // BB, nv // SG),
        in_specs=[
            pl.BlockSpec((BB, D), lambda b, t: (b, 0)),
            pl.BlockSpec((CH * SG, D), lambda b, t: (t, 0)),
        ],
        out_specs=[
            pl.BlockSpec((BB // 8, SG, 8, CH), lambda b, t: (b, t, 0, 0)),
            pl.BlockSpec((BB // 8, 1, 8, CH), lambda b, t: (b, t, 0, 0)),
        ],
        out_shape=[
            jax.ShapeDtypeStruct((B // 8, nv, 8, CH), jnp.float32),
            jax.ShapeDtypeStruct((B // 8, nv // SG, 8, CH), jnp.float32),
        ],
    )(qn, kn)

    num_cores, num_subcores = 2, 16
    nw = num_cores * num_subcores
    sc = _make_sc_topk(B, npad, n, B // nw, num_cores, num_subcores)
    sim2 = simv.reshape(B * npad // L, L)
    idx, out_keys, out_vals = sc(bmax, sim2, keys, vals)
    scores = jnp.zeros((B, K_TOP), dtype=jnp.float32)
    return (out_keys, out_vals, scores, idx)
